# baseline (speedup 1.0000x reference)
import jax
import jax.numpy as jnp
from jax import lax
from jax.experimental import pallas as pl
from jax.experimental.pallas import tpu as pltpu

_FRACS = (0.25, 0.5, 0.75, 0.9)


def _mask_kernel(zo_ref, x_ref, o0, o1, o2, o3, vs_ref, *, row_off, col_off,
                 tt):
    vs_ref[...] = x_ref[:, row_off, :]
    v = vs_ref[...]
    ones = jnp.ones(v.shape, jnp.int32)
    zeros = jnp.zeros(v.shape, jnp.int32)
    rank = zeros
    iota = lax.broadcasted_iota(jnp.int32, v.shape, 1)
    for j in range(col_off, col_off + tt):
        vj = v[:, j:j + 1]
        cmp = (vj < v) | ((vj == v) & (iota > j))
        rank = rank + jnp.where(cmp, ones, zeros)
    rank_w = rank[:, col_off:col_off + tt]
    zo = zo_ref[0, 0]
    for out, frac in zip((o0, o1, o2, o3), _FRACS):
        out[...] = rank_w >= int(tt * frac) + zo


def kernel(attn_avg, inference, template_token_num):
    B, S, L = attn_avg.shape
    T = L - 200
    tt = 49
    row = (S - 200) + (14 // 2) * 14 + 14 // 2
    col0 = T - tt
    r_blk = row // 8
    c_blk = col0 // 128
    assert col0 - c_blk * 128 + tt <= 128
    zero_offset = jnp.reshape(
        jnp.asarray(template_token_num, jnp.int32) - T, (1, 1))

    bb = B // 4
    out_sd = jax.ShapeDtypeStruct((B, tt), jnp.bool_)
    outs = pl.pallas_call(
        lambda zo, x, o0, o1, o2, o3, vs: _mask_kernel(
            zo, x, o0, o1, o2, o3, vs,
            row_off=row - r_blk * 8, col_off=col0 - c_blk * 128, tt=tt),
        grid=(4,),
        scratch_shapes=[pltpu.VMEM((bb, 128), jnp.float32)],
        in_specs=[
            pl.BlockSpec(memory_space=pltpu.SMEM),
            pl.BlockSpec((bb, 8, 128), lambda i: (i, r_blk, c_blk)),
        ],
        out_specs=[pl.BlockSpec((bb, tt), lambda i: (i, 0))] * 4,
        out_shape=[out_sd] * 4,
    )(zero_offset, attn_avg)
    return tuple(outs)


# speedup vs baseline: 1.0055x; 1.0055x over previous
import jax
import jax.numpy as jnp
from jax import lax
from jax.experimental import pallas as pl
from jax.experimental.pallas import tpu as pltpu

_FRACS = (0.25, 0.5, 0.75, 0.9)


def _mask_kernel(zo_ref, x_ref, o0, o1, o2, o3, vs_ref, *, row_off, col_off,
                 tt):
    vs_ref[...] = x_ref[:, row_off, :]
    v = vs_ref[...]
    ones = jnp.ones(v.shape, jnp.int32)
    zeros = jnp.zeros(v.shape, jnp.int32)
    rank = zeros
    iota = lax.broadcasted_iota(jnp.int32, v.shape, 1)
    for j in range(col_off, col_off + tt):
        vj = v[:, j:j + 1]
        cmp = (vj < v) | ((vj == v) & (iota > j))
        rank = rank + jnp.where(cmp, ones, zeros)
    rank_w = rank[:, col_off:col_off + tt]
    zo = zo_ref[0, 0]
    for out, frac in zip((o0, o1, o2, o3), _FRACS):
        out[...] = rank_w >= int(tt * frac) + zo


def kernel(attn_avg, inference, template_token_num):
    B, S, L = attn_avg.shape
    T = L - 200
    tt = 49
    row = (S - 200) + (14 // 2) * 14 + 14 // 2
    col0 = T - tt
    r_blk = row // 8
    c_blk = col0 // 128
    assert col0 - c_blk * 128 + tt <= 128
    zero_offset = jnp.reshape(
        jnp.asarray(template_token_num, jnp.int32) - T, (1, 1))

    out_sd = jax.ShapeDtypeStruct((B, tt), jnp.bool_)
    outs = pl.pallas_call(
        lambda zo, x, o0, o1, o2, o3, vs: _mask_kernel(
            zo, x, o0, o1, o2, o3, vs,
            row_off=row - r_blk * 8, col_off=col0 - c_blk * 128, tt=tt),
        grid=(1,),
        scratch_shapes=[pltpu.VMEM((B, 128), jnp.float32)],
        in_specs=[
            pl.BlockSpec(memory_space=pltpu.SMEM),
            pl.BlockSpec((B, 8, 128), lambda i: (0, r_blk, c_blk)),
        ],
        out_specs=[pl.BlockSpec((B, tt), lambda i: (0, 0))] * 4,
        out_shape=[out_sd] * 4,
    )(zero_offset, attn_avg)
    return tuple(outs)


# speedup vs baseline: 1.0083x; 1.0028x over previous
import jax
import jax.numpy as jnp
from jax import lax
from jax.experimental import pallas as pl
from jax.experimental.pallas import tpu as pltpu

_FRACS = (0.25, 0.5, 0.75, 0.9)


def _mask_kernel(zo_ref, x_ref, o0, o1, o2, o3, vs_ref, sem, *, row, col_al,
                 win_off, tt):
    pltpu.make_async_copy(
        x_ref.at[:, row, pl.ds(col_al, 128)], vs_ref, sem).start()
    pltpu.make_async_copy(
        x_ref.at[:, row, pl.ds(col_al, 128)], vs_ref, sem).wait()
    v = vs_ref[...]
    ones = jnp.ones(v.shape, jnp.int32)
    zeros = jnp.zeros(v.shape, jnp.int32)
    rank = zeros
    iota = lax.broadcasted_iota(jnp.int32, v.shape, 1)
    for j in range(win_off, win_off + tt):
        vj = v[:, j:j + 1]
        cmp = (vj < v) | ((vj == v) & (iota > j))
        rank = rank + jnp.where(cmp, ones, zeros)
    rank_w = rank[:, win_off:win_off + tt]
    zo = zo_ref[0, 0]
    for out, frac in zip((o0, o1, o2, o3), _FRACS):
        out[...] = rank_w >= int(tt * frac) + zo


def kernel(attn_avg, inference, template_token_num):
    B, S, L = attn_avg.shape
    T = L - 200
    tt = 49
    row = (S - 200) + (14 // 2) * 14 + 14 // 2
    col0 = T - tt
    col_al = (col0 // 128) * 128
    win_off = col0 - col_al
    zero_offset = jnp.reshape(
        jnp.asarray(template_token_num, jnp.int32) - T, (1, 1))

    out_sd = jax.ShapeDtypeStruct((B, tt), jnp.bool_)
    outs = pl.pallas_call(
        lambda zo, x, o0, o1, o2, o3, vs, sem: _mask_kernel(
            zo, x, o0, o1, o2, o3, vs, sem,
            row=row, col_al=col_al, win_off=win_off, tt=tt),
        grid=(1,),
        scratch_shapes=[pltpu.VMEM((B, 128), jnp.float32),
                        pltpu.SemaphoreType.DMA],
        in_specs=[
            pl.BlockSpec(memory_space=pltpu.SMEM),
            pl.BlockSpec(memory_space=pl.ANY),
        ],
        out_specs=[pl.BlockSpec((B, tt), lambda i: (0, 0))] * 4,
        out_shape=[out_sd] * 4,
    )(zero_offset, attn_avg)
    return tuple(outs)
